# baseline (speedup 1.0000x reference)
import functools

import jax
import jax.numpy as jnp
from jax import lax
from jax.experimental import pallas as pl
from jax.experimental.pallas import tpu as pltpu
from jax.experimental.pallas import tpu_sc as plsc

_ALPHA = 0.2
_NPAD = 1024
_CH = 112


def _sc_body(nc, ns, e_w, nch,
             xT2, idxg, idxs, wexp, wd, dsel, zx, zd, outx, outd,
             rows_v, msgd_v, idxg_v, idxs_v, wexp_v, wd_v, dsel_v,
             accx_s, accd_s, sem):
    cid = lax.axis_index("c")
    sid = lax.axis_index("s")
    wid = sid * nc + cid

    pltpu.sync_copy(idxg.at[wid], idxg_v)
    pltpu.sync_copy(idxs.at[wid], idxs_v)
    pltpu.sync_copy(wexp.at[wid], wexp_v)
    pltpu.sync_copy(wd.at[wid], wd_v)
    pltpu.sync_copy(dsel.at[wid], dsel_v)

    rpw = _NPAD // ns
    pltpu.sync_copy(zx, accx_s.at[pl.ds(sid * rpw, rpw)])
    pltpu.sync_copy(zd, accd_s.at[pl.ds(sid * rpw, rpw)])

    cps = [
        pltpu.async_copy(xT2.at[idxg_v.at[c]],
                         rows_v.at[pl.ds(c * _CH, _CH)], sem)
        for c in range(nch)
    ]
    for cp in cps:
        cp.wait()

    def mul_body(e, carry):
        for g in range(4):
            s = pl.ds(g * 16, 16)
            rows_v[e, s] = rows_v[e, s] * wexp_v[e, s]
        s0 = pl.ds(0, 16)
        msgd_v[e, s0] = dsel_v[e, s0] * wd_v[e, s0]
        return carry
    lax.fori_loop(0, e_w, mul_body, 0)

    plsc.subcore_barrier()

    for c in range(nch):
        pltpu.sync_copy(rows_v.at[pl.ds(c * _CH, _CH)],
                        accx_s.at[idxs_v.at[c]], add=True)
        pltpu.sync_copy(msgd_v.at[pl.ds(c * _CH, _CH)],
                        accd_s.at[idxs_v.at[c]], add=True)

    plsc.subcore_barrier()

    pltpu.sync_copy(accx_s.at[pl.ds(sid * rpw, rpw)],
                    outx.at[cid, pl.ds(sid * rpw, rpw)])
    pltpu.sync_copy(accd_s.at[pl.ds(sid * rpw, rpw)],
                    outd.at[cid, pl.ds(sid * rpw, rpw)])


def _fc_body(x_ref, d_ref, Wc_ref, bt_ref, out_ref):
    accx = x_ref[0, :, :] + x_ref[1, :, :]
    accd = d_ref[0, :, :] + d_ref[1, :, :]
    acc = jnp.concatenate([accx, accd], axis=1)
    out64 = jnp.dot(acc, Wc_ref[...],
                    preferred_element_type=jnp.float32) + bt_ref[0:1, :]
    parts = []
    for h in range(2):
        for t in range(4):
            base = 32 * h + 8 * t
            cur = out64[:, base:base + 8]
            if t == 0:
                parts.append(cur)
            else:
                prev = out64[:, base - 8:base]
                parts.append(_ALPHA * prev + (1.0 - _ALPHA) * cur)
    sm = jnp.concatenate(parts, axis=1)
    out_ref[...] = 1.0 / (1.0 + jnp.exp(-sm))


def kernel(x, T, d_ew, d_edges, d_dist, W, b):
    del T
    _, T_, N, Cx = x.shape
    E = d_edges.shape[0]
    F = T_ * Cx

    info = plsc.get_sparse_core_info()
    nc, ns = info.num_cores, info.num_subcores
    nw = nc * ns
    e_w = -(-E // (nw * _CH)) * _CH
    nch = e_w // _CH
    e_pad = nw * e_w

    xT = x[0].transpose(1, 0, 2).reshape(N, F)
    xT2 = jnp.zeros((_NPAD, 2 * F), jnp.float32)
    xT2 = xT2.at[:N, :F].set(xT).at[:N, F:].set(xT)

    pad_row = _NPAD - 8
    ni = jnp.full((e_pad,), pad_row, jnp.int32).at[:E].set(d_edges[:, 0])
    nj = jnp.full((e_pad,), pad_row, jnp.int32).at[:E].set(d_edges[:, 1])
    dew = jnp.zeros((e_pad, 2), jnp.float32).at[:E].set(d_ew)
    ddp = jnp.zeros((e_pad,), jnp.float32).at[:E].set(d_dist)

    idxg = ni.reshape(nw, nch, _CH)
    idxs = nj.reshape(nw, nch, _CH)
    wexp = jnp.repeat(dew, F, axis=1).reshape(nw, e_w, 2 * F)
    wd = jnp.zeros((e_pad, 16), jnp.float32).at[:, :2].set(dew)
    wd = wd.reshape(nw, e_w, 16)
    dsel = jnp.zeros((e_pad, 16), jnp.float32)
    dsel = dsel.at[:, 0].set(ddp).at[:, 1].set(ddp).reshape(nw, e_w, 16)
    rpw = _NPAD // ns
    zx = jnp.zeros((rpw, 2 * F), jnp.float32)
    zd = jnp.zeros((rpw, 16), jnp.float32)

    mesh = plsc.VectorSubcoreMesh(core_axis_name="c", subcore_axis_name="s")
    outx, outd = pl.kernel(
        functools.partial(_sc_body, nc, ns, e_w, nch),
        out_type=[
            jax.ShapeDtypeStruct((nc, _NPAD, 2 * F), jnp.float32),
            jax.ShapeDtypeStruct((nc, _NPAD, 16), jnp.float32),
        ],
        mesh=mesh,
        compiler_params=pltpu.CompilerParams(use_tc_tiling_on_sc=False),
        scratch_types=[
            pltpu.VMEM((e_w, 2 * F), jnp.float32),
            pltpu.VMEM((e_w, 16), jnp.float32),
            pltpu.VMEM((nch, _CH), jnp.int32),
            pltpu.VMEM((nch, _CH), jnp.int32),
            pltpu.VMEM((e_w, 2 * F), jnp.float32),
            pltpu.VMEM((e_w, 16), jnp.float32),
            pltpu.VMEM((e_w, 16), jnp.float32),
            pltpu.VMEM_SHARED((_NPAD, 2 * F), jnp.float32),
            pltpu.VMEM_SHARED((_NPAD, 16), jnp.float32),
            pltpu.SemaphoreType.DMA,
        ],
    )(xT2, idxg, idxs, wexp, wd, dsel, zx, zd)

    Wc = jnp.zeros((64 + 16, 64), jnp.float32)
    for h in range(2):
        for t in range(4):
            base = 32 * h + 8 * t
            Wc = Wc.at[base:base + 8, base:base + 8].set(W[:8, :])
    Wc = Wc.at[64, 0:32].set(jnp.tile(W[8, :], 4))
    Wc = Wc.at[65, 32:64].set(jnp.tile(W[8, :], 4))
    bt = jnp.broadcast_to(jnp.tile(b, 8)[None, :], (8, 64))

    out = pl.pallas_call(
        _fc_body,
        in_specs=[
            pl.BlockSpec((nc, _NPAD, 64), lambda: (0, 0, 0)),
            pl.BlockSpec((nc, _NPAD, 16), lambda: (0, 0, 0)),
            pl.BlockSpec((80, 64), lambda: (0, 0)),
            pl.BlockSpec((8, 64), lambda: (0, 0)),
        ],
        out_specs=pl.BlockSpec((_NPAD, 64), lambda: (0, 0)),
        out_shape=jax.ShapeDtypeStruct((_NPAD, 64), jnp.float32),
    )(outx, outd, Wc, bt)

    res = out[:N].reshape(N, 2, T_, 8).transpose(2, 0, 1, 3)
    return res[None]


# speedup vs baseline: 2.0060x; 2.0060x over previous
import functools

import jax
import jax.numpy as jnp
from jax import lax
from jax.experimental import pallas as pl
from jax.experimental.pallas import tpu as pltpu
from jax.experimental.pallas import tpu_sc as plsc

_ALPHA = 0.2
_NPAD = 1024
_CH = 112


def _sc_body(nc, ns, e_w, nch,
             xT, idxg, idxs, wflat, ddflat, zx, zd,
             outx0, outx1, outd,
             rg_v, r1_v, msgd_v, idxg_v, idxs_v, w_v, dd_v,
             acc0_s, acc1_s, accd_s, gsem, ssem):
    cid = lax.axis_index("c")
    sid = lax.axis_index("s")
    wid = sid * nc + cid

    pltpu.sync_copy(idxg.at[wid], idxg_v)
    pltpu.sync_copy(idxs.at[wid], idxs_v)
    pltpu.sync_copy(wflat.at[wid], w_v)
    pltpu.sync_copy(ddflat.at[wid], dd_v)

    rpw = _NPAD // ns
    pltpu.sync_copy(zx, acc0_s.at[pl.ds(sid * rpw, rpw)])
    pltpu.sync_copy(zx, acc1_s.at[pl.ds(sid * rpw, rpw)])
    pltpu.sync_copy(zd, accd_s.at[pl.ds(sid * rpw, rpw)])

    gcps = [
        pltpu.async_copy(xT.at[idxg_v.at[c]],
                         rg_v.at[pl.ds(c * _CH, _CH)], gsem)
        for c in range(nch)
    ]
    plsc.subcore_barrier()

    lane = lax.iota(jnp.int32, 16)
    m01 = lane == 0
    s0, s1 = pl.ds(0, 16), pl.ds(16, 16)
    scps = []
    for c in range(nch):
        gcps[c].wait()

        @plsc.parallel_loop(0, _CH, step=1, unroll=2)
        def mul_body(k, _c=c):
            e = _c * _CH + k
            w0b = plsc.load_gather(
                w_v, [jnp.full((16,), 2 * e, jnp.int32)])
            w1b = plsc.load_gather(
                w_v, [jnp.full((16,), 2 * e + 1, jnp.int32)])
            ddb = plsc.load_gather(dd_v, [jnp.full((16,), e, jnp.int32)])
            a = rg_v[e, s0]
            b_ = rg_v[e, s1]
            r1_v[e, s0] = w1b * a
            r1_v[e, s1] = w1b * b_
            rg_v[e, s0] = w0b * a
            rg_v[e, s1] = w0b * b_
            msgd_v[e, s0] = jnp.where(m01, w0b, w1b) * ddb

        scps.append(pltpu.async_copy(
            rg_v.at[pl.ds(c * _CH, _CH)], acc0_s.at[idxs_v.at[c]],
            ssem, add=True))
        scps.append(pltpu.async_copy(
            r1_v.at[pl.ds(c * _CH, _CH)], acc1_s.at[idxs_v.at[c]],
            ssem, add=True))
        scps.append(pltpu.async_copy(
            msgd_v.at[pl.ds(c * _CH, _CH)], accd_s.at[idxs_v.at[c]],
            ssem, add=True))
    for cp in scps:
        cp.wait()

    plsc.subcore_barrier()

    sl = pl.ds(sid * rpw, rpw)
    pltpu.sync_copy(acc0_s.at[sl], outx0.at[cid, sl])
    pltpu.sync_copy(acc1_s.at[sl], outx1.at[cid, sl])
    pltpu.sync_copy(accd_s.at[sl], outd.at[cid, sl])


def _fc_body(x0_ref, x1_ref, d_ref, Wc_ref, bt_ref, out_ref):
    a0 = x0_ref[0, :, :] + x0_ref[1, :, :]
    a1 = x1_ref[0, :, :] + x1_ref[1, :, :]
    ad = d_ref[0, :, :] + d_ref[1, :, :]
    acc = jnp.concatenate([a0, a1, ad], axis=1)
    out64 = jnp.dot(acc, Wc_ref[...],
                    preferred_element_type=jnp.float32) + bt_ref[0:1, :]
    parts = []
    for h in range(2):
        for t in range(4):
            base = 32 * h + 8 * t
            cur = out64[:, base:base + 8]
            if t == 0:
                parts.append(cur)
            else:
                prev = out64[:, base - 8:base]
                parts.append(_ALPHA * prev + (1.0 - _ALPHA) * cur)
    sm = jnp.concatenate(parts, axis=1)
    out_ref[...] = 1.0 / (1.0 + jnp.exp(-sm))


def kernel(x, T, d_ew, d_edges, d_dist, W, b):
    del T
    _, T_, N, Cx = x.shape
    E = d_edges.shape[0]
    F = T_ * Cx

    info = plsc.get_sparse_core_info()
    nc, ns = info.num_cores, info.num_subcores
    nw = nc * ns
    e_w = -(-E // (nw * _CH)) * _CH
    nch = e_w // _CH
    e_pad = nw * e_w

    xT = x[0].transpose(1, 0, 2).reshape(N, F)
    xTp = jnp.zeros((_NPAD, F), jnp.float32).at[:N].set(xT)

    pad_row = _NPAD - 8
    ni = jnp.full((e_pad,), pad_row, jnp.int32).at[:E].set(d_edges[:, 0])
    nj = jnp.full((e_pad,), pad_row, jnp.int32).at[:E].set(d_edges[:, 1])
    wfl = jnp.zeros((e_pad, 2), jnp.float32).at[:E].set(d_ew)
    ddp = jnp.zeros((e_pad,), jnp.float32).at[:E].set(d_dist)

    idxg = ni.reshape(nw, nch, _CH)
    idxs = nj.reshape(nw, nch, _CH)
    wflat = wfl.reshape(nw, e_w * 2)
    ddflat = ddp.reshape(nw, e_w)
    rpw = _NPAD // ns
    zx = jnp.zeros((rpw, F), jnp.float32)
    zd = jnp.zeros((rpw, 16), jnp.float32)

    mesh = plsc.VectorSubcoreMesh(core_axis_name="c", subcore_axis_name="s")
    outx0, outx1, outd = pl.kernel(
        functools.partial(_sc_body, nc, ns, e_w, nch),
        out_type=[
            jax.ShapeDtypeStruct((nc, _NPAD, F), jnp.float32),
            jax.ShapeDtypeStruct((nc, _NPAD, F), jnp.float32),
            jax.ShapeDtypeStruct((nc, _NPAD, 16), jnp.float32),
        ],
        mesh=mesh,
        compiler_params=pltpu.CompilerParams(
            use_tc_tiling_on_sc=False, needs_layout_passes=False),
        scratch_types=[
            pltpu.VMEM((e_w, F), jnp.float32),
            pltpu.VMEM((e_w, F), jnp.float32),
            pltpu.VMEM((e_w, 16), jnp.float32),
            pltpu.VMEM((nch, _CH), jnp.int32),
            pltpu.VMEM((nch, _CH), jnp.int32),
            pltpu.VMEM((e_w * 2,), jnp.float32),
            pltpu.VMEM((e_w,), jnp.float32),
            pltpu.VMEM_SHARED((_NPAD, F), jnp.float32),
            pltpu.VMEM_SHARED((_NPAD, F), jnp.float32),
            pltpu.VMEM_SHARED((_NPAD, 16), jnp.float32),
            pltpu.SemaphoreType.DMA,
            pltpu.SemaphoreType.DMA,
        ],
    )(xTp, idxg, idxs, wflat, ddflat, zx, zd)

    Wc = jnp.zeros((64 + 16, 64), jnp.float32)
    for h in range(2):
        for t in range(4):
            base = 32 * h + 8 * t
            Wc = Wc.at[base:base + 8, base:base + 8].set(W[:8, :])
    Wc = Wc.at[64, 0:32].set(jnp.tile(W[8, :], 4))
    Wc = Wc.at[65, 32:64].set(jnp.tile(W[8, :], 4))
    bt = jnp.broadcast_to(jnp.tile(b, 8)[None, :], (8, 64))

    out = pl.pallas_call(
        _fc_body,
        in_specs=[
            pl.BlockSpec((nc, _NPAD, F), lambda: (0, 0, 0)),
            pl.BlockSpec((nc, _NPAD, F), lambda: (0, 0, 0)),
            pl.BlockSpec((nc, _NPAD, 16), lambda: (0, 0, 0)),
            pl.BlockSpec((80, 64), lambda: (0, 0)),
            pl.BlockSpec((8, 64), lambda: (0, 0)),
        ],
        out_specs=pl.BlockSpec((_NPAD, 64), lambda: (0, 0)),
        out_shape=jax.ShapeDtypeStruct((_NPAD, 64), jnp.float32),
    )(outx0, outx1, outd, Wc, bt)

    res = out[:N].reshape(N, 2, T_, 8).transpose(2, 0, 1, 3)
    return res[None]
